# baseline (speedup 1.0000x reference)
import jax
import jax.numpy as jnp
from jax import lax
from jax.experimental import pallas as pl
from jax.experimental.pallas import tpu as pltpu
from jax.experimental.pallas import tpu_sc as plsc

_NC = 2
_NS = 16
_NW = _NC * _NS
_D = 32
_L = 16
_CHUNK = 512
_U = 16


def _make_body(b_per_w):
    nch = b_per_w // _CHUNK
    cw = _CHUNK * _D

    def _emb_body(idx_hbm, table_hbm, out_hbm, tab_v, idx_v, obuf, wsem):
        wid = lax.axis_index("s") * _NC + lax.axis_index("c")
        base = wid * b_per_w
        pltpu.sync_copy(table_hbm, tab_v)
        pltpu.sync_copy(idx_hbm.at[pl.ds(base, b_per_w)], idx_v)

        def _write_args(j):
            return (
                obuf.at[j % 2],
                out_hbm.at[pl.ds((base + j * _CHUNK) * _D, cw)],
                wsem,
            )

        for j in range(nch):
            slot = j % 2
            if j >= 2:
                pltpu.make_async_copy(*_write_args(j - 2)).wait()

            @plsc.parallel_loop(0, _CHUNK // _U, unroll=2)
            def _(g):
                r0 = g * _U
                off16 = idx_v[pl.ds(j * _CHUNK + r0, _U)]
                for u in range(_U):
                    off = off16[u]
                    v0 = tab_v[pl.ds(off, _L)]
                    v1 = tab_v[pl.ds(off + _L, _L)]
                    obuf[slot, pl.ds((r0 + u) * _D, _L)] = v0
                    obuf[slot, pl.ds((r0 + u) * _D + _L, _L)] = v1

            pltpu.async_copy(*_write_args(j))

        for j in range(max(nch - 2, 0), nch):
            pltpu.make_async_copy(*_write_args(j)).wait()

    return _emb_body


_TCBLK = 1024
_SC_NUM = 9
_SC_DEN = 16


def _tc_body(idx_ref, prj_ref, tab_ref, out_ref):
    idxf = idx_ref[...].astype(jnp.float32)
    idx_rep = jnp.dot(idxf, prj_ref[...], preferred_element_type=jnp.float32)
    j64 = lax.broadcasted_iota(jnp.int32, (_TCBLK, 4 * _L), 1)
    vals = ((j64 % _L) * _D).astype(jnp.float32)
    oh = (idx_rep == vals).astype(jnp.float32)
    out_ref[...] = jnp.dot(
        oh, tab_ref[...], preferred_element_type=jnp.float32,
    )


def kernel(land_use_map, table):
    H, W = land_use_map.shape
    V, D = table.shape
    B = H * W
    B_sc = B * _SC_NUM // _SC_DEN
    assert D == _D and B_sc % (_NW * _CHUNK) == 0 and (B - B_sc) % _TCBLK == 0
    b_per_w = B_sc // _NW
    idx = land_use_map.astype(jnp.int32).reshape(B) * _D
    tab_flat = table.reshape(V * D)

    fn = pl.kernel(
        _make_body(b_per_w),
        out_type=jax.ShapeDtypeStruct((B_sc * D,), jnp.float32),
        mesh=plsc.VectorSubcoreMesh(core_axis_name="c", subcore_axis_name="s"),
        scratch_types=[
            pltpu.VMEM((V * D,), jnp.float32),
            pltpu.VMEM((b_per_w,), jnp.int32),
            pltpu.VMEM((2, _CHUNK * _D), jnp.float32),
            pltpu.SemaphoreType.DMA,
        ],
        compiler_params=pltpu.CompilerParams(use_tc_tiling_on_sc=False),
    )
    out_sc = fn(idx[:B_sc], tab_flat)
    B_tc = B - B_sc
    idx4 = idx[B_sc:].reshape(B_tc // 4, 4)
    prj = (lax.broadcasted_iota(jnp.int32, (4, 64), 1) // _L ==
           lax.broadcasted_iota(jnp.int32, (4, 64), 0)).astype(jnp.float32)
    tab64 = jnp.zeros((4, _L, 4, D), jnp.float32)
    tab64 = tab64.at[jnp.arange(4), :, jnp.arange(4), :].set(
        jnp.zeros((_L, D), jnp.float32).at[:V].set(table)
    ).reshape(64, 4 * D)
    nblk = B_tc // 4 // _TCBLK
    out_tc = pl.pallas_call(
        _tc_body,
        grid=(nblk,),
        in_specs=[
            pl.BlockSpec((_TCBLK, 4), lambda i: (i, 0)),
            pl.BlockSpec((4, 64), lambda i: (0, 0)),
            pl.BlockSpec((64, 4 * D), lambda i: (0, 0)),
        ],
        out_specs=pl.BlockSpec((_TCBLK, 4 * D), lambda i: (i, 0)),
        out_shape=jax.ShapeDtypeStruct((B_tc // 4, 4 * D), jnp.float32),
    )(idx4, prj, tab64)
    out = jnp.concatenate([out_sc, out_tc.reshape(B_tc * D)])
    return out.reshape(H, W, D)


# speedup vs baseline: 1.4234x; 1.4234x over previous
import jax
import jax.numpy as jnp
from jax import lax
from jax.experimental import pallas as pl
from jax.experimental.pallas import tpu as pltpu
from jax.experimental.pallas import tpu_sc as plsc

_NC = 2
_NS = 16
_NW = _NC * _NS
_D = 32
_L = 16
_CHUNK = 512
_U = 16


def _make_body(b_per_w):
    nch = b_per_w // _CHUNK
    cw = _CHUNK * _D

    def _emb_body(idx_hbm, table_hbm, out_hbm, tab_v, idx_v, obuf, wsem):
        wid = lax.axis_index("s") * _NC + lax.axis_index("c")
        base = wid * b_per_w
        pltpu.sync_copy(table_hbm, tab_v)
        pltpu.sync_copy(idx_hbm.at[pl.ds(base, b_per_w)], idx_v)

        def _write_args(j):
            return (
                obuf.at[j % 2],
                out_hbm.at[pl.ds((base + j * _CHUNK) * _D, cw)],
                wsem,
            )

        for j in range(nch):
            slot = j % 2
            if j >= 2:
                pltpu.make_async_copy(*_write_args(j - 2)).wait()

            @plsc.parallel_loop(0, _CHUNK // _U, unroll=2)
            def _(g):
                r0 = g * _U
                off16 = idx_v[pl.ds(j * _CHUNK + r0, _U)]
                for u in range(_U):
                    off = off16[u]
                    v0 = tab_v[pl.ds(off, _L)]
                    v1 = tab_v[pl.ds(off + _L, _L)]
                    obuf[slot, pl.ds((r0 + u) * _D, _L)] = v0
                    obuf[slot, pl.ds((r0 + u) * _D + _L, _L)] = v1

            pltpu.async_copy(*_write_args(j))

        for j in range(max(nch - 2, 0), nch):
            pltpu.make_async_copy(*_write_args(j)).wait()

    return _emb_body


def kernel(land_use_map, table):
    H, W = land_use_map.shape
    V, D = table.shape
    B = H * W
    assert D == _D and B % (_NW * _CHUNK) == 0
    b_per_w = B // _NW
    idx = land_use_map.astype(jnp.int32).reshape(B) * _D
    tab_flat = table.reshape(V * D)

    fn = pl.kernel(
        _make_body(b_per_w),
        out_type=jax.ShapeDtypeStruct((B * D,), jnp.float32),
        mesh=plsc.VectorSubcoreMesh(core_axis_name="c", subcore_axis_name="s"),
        scratch_types=[
            pltpu.VMEM((V * D,), jnp.float32),
            pltpu.VMEM((b_per_w,), jnp.int32),
            pltpu.VMEM((2, _CHUNK * _D), jnp.float32),
            pltpu.SemaphoreType.DMA,
        ],
        compiler_params=pltpu.CompilerParams(use_tc_tiling_on_sc=False),
    )
    return fn(idx, tab_flat).reshape(H, W, D)


# speedup vs baseline: 1.4702x; 1.0329x over previous
import jax
import jax.numpy as jnp
from jax import lax
from jax.experimental import pallas as pl
from jax.experimental.pallas import tpu as pltpu
from jax.experimental.pallas import tpu_sc as plsc

_NC = 2
_NS = 16
_NW = _NC * _NS
_D = 32
_L = 16
_CHUNK = 1024
_U = 16


def _make_body(b_per_w):
    nch = b_per_w // _CHUNK
    cw = _CHUNK * _D

    def _emb_body(idx_hbm, table_hbm, out_hbm, tab_v, idx_v, obuf, wsem):
        wid = lax.axis_index("s") * _NC + lax.axis_index("c")
        base = wid * b_per_w
        pltpu.sync_copy(table_hbm, tab_v)
        pltpu.sync_copy(idx_hbm.at[pl.ds(base, b_per_w)], idx_v)

        def _write_args(j):
            return (
                obuf.at[j % 2],
                out_hbm.at[pl.ds((base + j * _CHUNK) * _D, cw)],
                wsem,
            )

        for j in range(nch):
            slot = j % 2
            if j >= 2:
                pltpu.make_async_copy(*_write_args(j - 2)).wait()

            @plsc.parallel_loop(0, _CHUNK // _U, unroll=2)
            def _(g):
                r0 = g * _U
                off16 = idx_v[pl.ds(j * _CHUNK + r0, _U)]
                for u in range(_U):
                    off = off16[u]
                    v0 = tab_v[pl.ds(off, _L)]
                    v1 = tab_v[pl.ds(off + _L, _L)]
                    obuf[slot, pl.ds((r0 + u) * _D, _L)] = v0
                    obuf[slot, pl.ds((r0 + u) * _D + _L, _L)] = v1

            pltpu.async_copy(*_write_args(j))

        for j in range(max(nch - 2, 0), nch):
            pltpu.make_async_copy(*_write_args(j)).wait()

    return _emb_body


def kernel(land_use_map, table):
    H, W = land_use_map.shape
    V, D = table.shape
    B = H * W
    assert D == _D and B % (_NW * _CHUNK) == 0
    b_per_w = B // _NW
    idx = land_use_map.astype(jnp.int32).reshape(B) * _D
    tab_flat = table.reshape(V * D)

    fn = pl.kernel(
        _make_body(b_per_w),
        out_type=jax.ShapeDtypeStruct((B * D,), jnp.float32),
        mesh=plsc.VectorSubcoreMesh(core_axis_name="c", subcore_axis_name="s"),
        scratch_types=[
            pltpu.VMEM((V * D,), jnp.float32),
            pltpu.VMEM((b_per_w,), jnp.int32),
            pltpu.VMEM((2, _CHUNK * _D), jnp.float32),
            pltpu.SemaphoreType.DMA,
        ],
        compiler_params=pltpu.CompilerParams(use_tc_tiling_on_sc=False),
    )
    return fn(idx, tab_flat).reshape(H, W, D)
